# Initial kernel scaffold; baseline (speedup 1.0000x reference)
import jax
import jax.numpy as jnp
from jax.experimental import pallas as pl


def kernel(x_nchw, w1, b1, w2, b2):
    raise NotImplementedError("write your pallas kernel here")


# speedup vs baseline: 1.0972x; 1.0972x over previous
import jax
import jax.numpy as jnp
from jax.experimental import pallas as pl
from jax.experimental.pallas import tpu as pltpu


def _se_fused_kernel(x_ref, w1_ref, b1_ref, w2_ref, b2_ref, o_ref, *, inv_hw):
    x = x_ref[...]
    pooled = jnp.sum(x, axis=1, keepdims=True) * inv_hw
    h = jax.lax.dot_general(
        pooled, w1_ref[...], (((0,), (0,)), ((), ())),
        preferred_element_type=jnp.float32)
    h = jax.nn.relu(h + b1_ref[...])
    g = jax.lax.dot_general(
        w2_ref[...], h, (((0,), (1,)), ((), ())),
        preferred_element_type=jnp.float32)
    gate = jax.nn.sigmoid(g + b2_ref[...])
    o_ref[...] = x * gate


def kernel(x_nchw, w1, b1, w2, b2):
    B, C, H, W = x_nchw.shape
    HW = H * W
    Cr = w1.shape[1]

    x2d = x_nchw.reshape(B * C, HW)
    b2_col = b2.reshape(C, 1)

    out = pl.pallas_call(
        lambda *refs: _se_fused_kernel(*refs, inv_hw=1.0 / HW),
        out_shape=jax.ShapeDtypeStruct((B * C, HW), x2d.dtype),
        grid=(B,),
        in_specs=[
            pl.BlockSpec((C, HW), lambda i: (i, 0)),
            pl.BlockSpec((C, Cr), lambda i: (0, 0)),
            pl.BlockSpec((1, Cr), lambda i: (0, 0)),
            pl.BlockSpec((Cr, C), lambda i: (0, 0)),
            pl.BlockSpec((C, 1), lambda i: (0, 0)),
        ],
        out_specs=pl.BlockSpec((C, HW), lambda i: (i, 0)),
        compiler_params=pltpu.CompilerParams(
            dimension_semantics=("parallel",),
            vmem_limit_bytes=64 * 1024 * 1024),
    )(x2d, w1, b1, w2, b2_col)

    return out.reshape(B, C, H, W)


# speedup vs baseline: 1.3223x; 1.2052x over previous
import jax
import jax.numpy as jnp
from jax.experimental import pallas as pl
from jax.experimental.pallas import tpu as pltpu


def _se_kernel(x_ref, w1_ref, b1_ref, w2_ref, b2_ref, o_ref, *, inv_hw):
    x = x_ref[...]
    pooled = jnp.sum(x, axis=(2, 3)) * inv_hw
    h = jax.nn.relu(
        jnp.dot(pooled, w1_ref[...], preferred_element_type=jnp.float32)
        + b1_ref[...])
    g = jax.nn.sigmoid(
        jnp.dot(h, w2_ref[...], preferred_element_type=jnp.float32)
        + b2_ref[...])
    o_ref[...] = x * g[:, :, None, None]


def kernel(x_nchw, w1, b1, w2, b2):
    B, C, H, W = x_nchw.shape
    Cr = w1.shape[1]

    return pl.pallas_call(
        lambda *refs: _se_kernel(*refs, inv_hw=1.0 / (H * W)),
        out_shape=jax.ShapeDtypeStruct((B, C, H, W), x_nchw.dtype),
        grid=(B,),
        in_specs=[
            pl.BlockSpec((1, C, H, W), lambda i: (i, 0, 0, 0)),
            pl.BlockSpec((C, Cr), lambda i: (0, 0)),
            pl.BlockSpec((1, Cr), lambda i: (0, 0)),
            pl.BlockSpec((Cr, C), lambda i: (0, 0)),
            pl.BlockSpec((1, C), lambda i: (0, 0)),
        ],
        out_specs=pl.BlockSpec((1, C, H, W), lambda i: (i, 0, 0, 0)),
        compiler_params=pltpu.CompilerParams(
            dimension_semantics=("parallel",),
            vmem_limit_bytes=100 * 1024 * 1024),
    )(x_nchw, w1, b1, w2, b2)


# speedup vs baseline: 8.9923x; 6.8006x over previous
import jax
import jax.numpy as jnp
from jax.experimental import pallas as pl
from jax.experimental.pallas import tpu as pltpu


def _se_kernel(x_ref, w1_ref, b1_ref, w2_ref, b2_ref, o_ref, *, inv_hw):
    x = x_ref[...]
    pooled = jnp.sum(x, axis=(1, 2)) * inv_hw
    h = jax.nn.relu(
        jnp.dot(pooled, w1_ref[...], preferred_element_type=jnp.float32)
        + b1_ref[...])
    g = jax.nn.sigmoid(
        jnp.dot(h, w2_ref[...], preferred_element_type=jnp.float32)
        + b2_ref[...])
    o_ref[...] = x * g[:, None, None, :]


def kernel(x_nchw, w1, b1, w2, b2):
    B, C, H, W = x_nchw.shape
    Cr = w1.shape[1]

    x_nhwc = jnp.transpose(x_nchw, (0, 2, 3, 1))

    out = pl.pallas_call(
        lambda *refs: _se_kernel(*refs, inv_hw=1.0 / (H * W)),
        out_shape=jax.ShapeDtypeStruct((B, H, W, C), x_nchw.dtype),
        grid=(B,),
        in_specs=[
            pl.BlockSpec((1, H, W, C), lambda i: (i, 0, 0, 0)),
            pl.BlockSpec((C, Cr), lambda i: (0, 0)),
            pl.BlockSpec((1, Cr), lambda i: (0, 0)),
            pl.BlockSpec((Cr, C), lambda i: (0, 0)),
            pl.BlockSpec((1, C), lambda i: (0, 0)),
        ],
        out_specs=pl.BlockSpec((1, H, W, C), lambda i: (i, 0, 0, 0)),
        compiler_params=pltpu.CompilerParams(
            dimension_semantics=("parallel",),
            vmem_limit_bytes=100 * 1024 * 1024),
    )(x_nhwc, w1, b1, w2, b2)

    return jnp.transpose(out, (0, 3, 1, 2))


# speedup vs baseline: 9.3876x; 1.0440x over previous
import jax
import jax.numpy as jnp
from jax.experimental import pallas as pl
from jax.experimental.pallas import tpu as pltpu


def _se_kernel(x_ref, w1_ref, b1_ref, w2_ref, b2_ref, o_ref, *, inv_hw):
    x = x_ref[...]
    pooled = jnp.sum(x, axis=(1, 2)) * inv_hw
    h = jax.nn.relu(
        jnp.dot(pooled, w1_ref[...], preferred_element_type=jnp.float32)
        + b1_ref[...])
    g = jax.nn.sigmoid(
        jnp.dot(h, w2_ref[...], preferred_element_type=jnp.float32)
        + b2_ref[...])
    o_ref[...] = x * g[:, None, None, :]


def kernel(x_nchw, w1, b1, w2, b2):
    B, C, H, W = x_nchw.shape
    Cr = w1.shape[1]

    x_nhwc = jnp.transpose(x_nchw, (0, 2, 3, 1))

    BB = 2
    out = pl.pallas_call(
        lambda *refs: _se_kernel(*refs, inv_hw=1.0 / (H * W)),
        out_shape=jax.ShapeDtypeStruct((B, H, W, C), x_nchw.dtype),
        grid=(B // BB,),
        in_specs=[
            pl.BlockSpec((BB, H, W, C), lambda i: (i, 0, 0, 0)),
            pl.BlockSpec((C, Cr), lambda i: (0, 0)),
            pl.BlockSpec((1, Cr), lambda i: (0, 0)),
            pl.BlockSpec((Cr, C), lambda i: (0, 0)),
            pl.BlockSpec((1, C), lambda i: (0, 0)),
        ],
        out_specs=pl.BlockSpec((BB, H, W, C), lambda i: (i, 0, 0, 0)),
        compiler_params=pltpu.CompilerParams(
            dimension_semantics=("parallel",),
            vmem_limit_bytes=100 * 1024 * 1024),
    )(x_nhwc, w1, b1, w2, b2)

    return jnp.transpose(out, (0, 3, 1, 2))


# speedup vs baseline: 9.5676x; 1.0192x over previous
import jax
import jax.numpy as jnp
from jax.experimental import pallas as pl
from jax.experimental.pallas import tpu as pltpu


def _se_kernel(x_ref, w1t_ref, b1_ref, w2_ref, b2_ref, o_ref, *, inv_hw):
    x = x_ref[...]
    pooled = jnp.sum(x, axis=(1, 2)) * inv_hw
    h = jax.nn.relu(
        jax.lax.dot_general(
            pooled, w1t_ref[...], (((1,), (1,)), ((), ())),
            preferred_element_type=jnp.float32)
        + b1_ref[...])
    g = jax.nn.sigmoid(
        jnp.dot(h, w2_ref[...], preferred_element_type=jnp.float32)
        + b2_ref[...])
    o_ref[...] = x * g[:, None, None, :]


def kernel(x_nchw, w1, b1, w2, b2):
    B, C, H, W = x_nchw.shape
    Cr = w1.shape[1]

    x_nhwc = jnp.transpose(x_nchw, (0, 2, 3, 1))
    w1t = jnp.transpose(w1)

    BB = 2
    out = pl.pallas_call(
        lambda *refs: _se_kernel(*refs, inv_hw=1.0 / (H * W)),
        out_shape=jax.ShapeDtypeStruct((B, H, W, C), x_nchw.dtype),
        grid=(B // BB,),
        in_specs=[
            pl.BlockSpec((BB, H, W, C), lambda i: (i, 0, 0, 0)),
            pl.BlockSpec((Cr, C), lambda i: (0, 0)),
            pl.BlockSpec((1, Cr), lambda i: (0, 0)),
            pl.BlockSpec((Cr, C), lambda i: (0, 0)),
            pl.BlockSpec((1, C), lambda i: (0, 0)),
        ],
        out_specs=pl.BlockSpec((BB, H, W, C), lambda i: (i, 0, 0, 0)),
        compiler_params=pltpu.CompilerParams(
            dimension_semantics=("parallel",),
            vmem_limit_bytes=100 * 1024 * 1024),
    )(x_nhwc, w1t, b1, w2, b2)

    return jnp.transpose(out, (0, 3, 1, 2))
